# Initial kernel scaffold; baseline (speedup 1.0000x reference)
import jax
import jax.numpy as jnp
from jax.experimental import pallas as pl


def kernel(x_user, x_item, edge_index_ui, edge_index_iu, W_l0_ui, b_l0_ui, W_r0_ui, W_l0_iu, b_l0_iu, W_r0_iu, W_l1_ui, b_l1_ui, W_r1_ui, W_l1_iu, b_l1_iu, W_r1_iu, W_lin_user, b_lin_user, W_lin_item, b_lin_item):
    raise NotImplementedError("write your pallas kernel here")


# speedup vs baseline: 3.5494x; 3.5494x over previous
import functools

import jax
import jax.numpy as jnp
from jax import lax
from jax.experimental import pallas as pl
from jax.experimental.pallas import tpu as pltpu
from jax.experimental.pallas import tpu_sc as plsc

N = 10000
E = 320000
D_IN = 128
H = 64
D_EMB = 128

NC = 1
NS = 16
NW = NC * NS
CHUNK = 128
G = 1
WCH = 16
W = 128
EPT = -(-E // NW)
NCH = -(-(-(-EPT // CHUNK)) // WCH) * WCH
EPT_PAD = NCH * CHUNK
E_PAD = EPT_PAD * NW
N_PAD = 10240
RPS = N_PAD // NS

GRID = 16
BR = N_PAD // GRID

_f32 = jnp.float32


def _dot_t(a, b):
    return lax.dot_general(a, b, (((1,), (1,)), ((), ())),
                           preferred_element_type=_f32)


def _tc1_body(xu_ref, xi_ref, wui_ref, wiu_ref, yu_ref, yi_ref):
    marker = jnp.where(
        lax.broadcasted_iota(jnp.int32, (BR, W - H), 1) == 0, 1.0, 0.0
    ).astype(_f32)
    yu_ref[...] = jnp.concatenate([_dot_t(xu_ref[...], wui_ref[...]), marker], 1)
    yi_ref[...] = jnp.concatenate([_dot_t(xi_ref[...], wiu_ref[...]), marker], 1)


def _tc2_half(p, x, wr0, bl0, wr1, bl1):
    s = p[:, :H]
    cnt = p[:, H:H + 1]
    r = _dot_t(x, wr0) + bl0
    h = jnp.maximum(s / jnp.maximum(cnt, 1.0) + r, 0.0)
    t = _dot_t(h, wr1) + bl1
    hw = jnp.concatenate([h, jnp.zeros((h.shape[0], W - H), _f32)], 1)
    return hw, t


def _tc2_body(pi_ref, pu_ref, xi_ref, xu_ref,
              wr0ui_ref, bl0ui_ref, wr0iu_ref, bl0iu_ref,
              wr1ui_ref, bl1ui_ref, wr1iu_ref, bl1iu_ref,
              hi_ref, hu_ref, ti_ref, tu_ref):
    hi_ref[...], ti_ref[...] = _tc2_half(
        pi_ref[...], xi_ref[...], wr0ui_ref[...], bl0ui_ref[...],
        wr1ui_ref[...], bl1ui_ref[...])
    hu_ref[...], tu_ref[...] = _tc2_half(
        pu_ref[...], xu_ref[...], wr0iu_ref[...], bl0iu_ref[...],
        wr1iu_ref[...], bl1iu_ref[...])


def _tc3_half(p1, p0, t, wl1, wlin, blin):
    s1 = p1[:, :H]
    cnt = p0[:, H:H + 1]
    z = _dot_t(s1 / jnp.maximum(cnt, 1.0), wl1) + t
    return _dot_t(z, wlin) + blin


def _tc3_body(p1i_ref, p1u_ref, p0i_ref, p0u_ref, ti_ref, tu_ref,
              wl1ui_ref, wl1iu_ref, wlini_ref, blini_ref,
              wlinu_ref, blinu_ref, oi_ref, ou_ref):
    oi_ref[...] = _tc3_half(p1i_ref[...], p0i_ref[...], ti_ref[...],
                            wl1ui_ref[...], wlini_ref[...], blini_ref[...])
    ou_ref[...] = _tc3_half(p1u_ref[...], p0u_ref[...], tu_ref[...],
                            wl1iu_ref[...], wlinu_ref[...], blinu_ref[...])


def _row_spec(w):
    return pl.BlockSpec((BR, w), lambda i: (i, 0))


def _full_spec(shape):
    nd = len(shape)
    return pl.BlockSpec(shape, lambda i, _n=nd: (0,) * _n)


def _make_seg_sum():
    mesh = plsc.VectorSubcoreMesh(core_axis_name="c", subcore_axis_name="s",
                                  num_cores=NC, num_subcores=NS)
    NWIN = NCH // WCH
    NB = WCH // G

    @functools.partial(
        pl.kernel,
        out_type=jax.ShapeDtypeStruct((N_PAD, W), _f32),
        mesh=mesh,
        scratch_types=(
            pltpu.VMEM((WCH, CHUNK), jnp.int32),
            pltpu.VMEM((WCH, CHUNK), jnp.int32),
            [pltpu.VMEM((CHUNK, W), _f32) for _ in range(G)],
            pltpu.VMEM_SHARED((N_PAD, W), _f32),
            pltpu.SemaphoreType.DMA,
            pltpu.SemaphoreType.DMA,
        ),
    )
    def seg(tab, src3, dst3, out, srci, dsti, bufs, acc, gsem, ssem):
        sid = lax.axis_index("s")
        wid = sid

        def _zrow(i, carry):
            for c in range(W // 16):
                bufs[0][i, pl.ds(c * 16, 16)] = jnp.zeros((16,), _f32)
            return carry
        lax.fori_loop(0, CHUNK, _zrow, 0)
        for r in range(RPS // CHUNK):
            off = sid * RPS + r * CHUNK
            pltpu.sync_copy(bufs[0], acc.at[pl.ds(off, CHUNK)])
        plsc.subcore_barrier()

        def win(v, carry):
            pltpu.sync_copy(src3.at[wid, pl.ds(v * WCH, WCH)], srci)
            pltpu.sync_copy(dst3.at[wid, pl.ds(v * WCH, WCH)], dsti)

            def grp(g, c2):
                b = g * G
                for q in range(G):
                    pltpu.sync_copy(tab.at[srci.at[b + q]], bufs[q])
                    pltpu.sync_copy(bufs[q], acc.at[dsti.at[b + q]], add=True)
                return c2
            lax.fori_loop(0, NB, grp, 0)
            return carry
        lax.fori_loop(0, NWIN, win, 0)
        plsc.subcore_barrier()

        for r in range(RPS // CHUNK):
            off = sid * RPS + r * CHUNK
            pltpu.sync_copy(acc.at[pl.ds(off, CHUNK)],
                            out.at[pl.ds(off, CHUNK)])

    return seg


_seg_sum_cache = {}


def _seg_sum(tab, src3, dst3):
    if "k" not in _seg_sum_cache:
        _seg_sum_cache["k"] = _make_seg_sum()
    return _seg_sum_cache["k"](tab, src3, dst3)


def _prep_edges(ei):
    src = ei[0].astype(jnp.int32)
    dst = ei[1].astype(jnp.int32)
    npad = E_PAD - E
    padv = N + (jnp.arange(npad, dtype=jnp.int32) % (N_PAD - N))
    src_p = jnp.concatenate([src, padv]).reshape(NW, NCH, CHUNK)
    dst_p = jnp.concatenate([dst, padv]).reshape(NW, NCH, CHUNK)
    return src_p, dst_p


def _pad_rows(x):
    return jnp.pad(x, ((0, N_PAD - N), (0, 0)))


@jax.jit
def kernel(x_user, x_item, edge_index_ui, edge_index_iu,
           W_l0_ui, b_l0_ui, W_r0_ui, W_l0_iu, b_l0_iu, W_r0_iu,
           W_l1_ui, b_l1_ui, W_r1_ui, W_l1_iu, b_l1_iu, W_r1_iu,
           W_lin_user, b_lin_user, W_lin_item, b_lin_item):
    src_ui, dst_ui = _prep_edges(edge_index_ui)
    src_iu, dst_iu = _prep_edges(edge_index_iu)
    xu = _pad_rows(x_user)
    xi = _pad_rows(x_item)
    b2 = lambda b: b.reshape(1, -1)

    yu, yi = pl.pallas_call(
        _tc1_body,
        grid=(GRID,),
        in_specs=[_row_spec(D_IN), _row_spec(D_IN),
                  _full_spec((H, D_IN)), _full_spec((H, D_IN))],
        out_specs=[_row_spec(W), _row_spec(W)],
        out_shape=[jax.ShapeDtypeStruct((N_PAD, W), _f32)] * 2,
    )(xu, xi, W_l0_ui, W_l0_iu)

    p_item = _seg_sum(yu, src_ui, dst_ui)
    p_user = _seg_sum(yi, src_iu, dst_iu)

    hi, hu, ti, tu = pl.pallas_call(
        _tc2_body,
        grid=(GRID,),
        in_specs=[_row_spec(W), _row_spec(W),
                  _row_spec(D_IN), _row_spec(D_IN),
                  _full_spec((H, D_IN)), _full_spec((1, H)),
                  _full_spec((H, D_IN)), _full_spec((1, H)),
                  _full_spec((D_EMB, H)), _full_spec((1, D_EMB)),
                  _full_spec((D_EMB, H)), _full_spec((1, D_EMB))],
        out_specs=[_row_spec(W), _row_spec(W),
                   _row_spec(D_EMB), _row_spec(D_EMB)],
        out_shape=[jax.ShapeDtypeStruct((N_PAD, W), _f32)] * 2 +
                  [jax.ShapeDtypeStruct((N_PAD, D_EMB), _f32)] * 2,
    )(p_item, p_user, xi, xu,
      W_r0_ui, b2(b_l0_ui), W_r0_iu, b2(b_l0_iu),
      W_r1_ui, b2(b_l1_ui), W_r1_iu, b2(b_l1_iu))

    p1_item = _seg_sum(hu, src_ui, dst_ui)
    p1_user = _seg_sum(hi, src_iu, dst_iu)

    oi, ou = pl.pallas_call(
        _tc3_body,
        grid=(GRID,),
        in_specs=[_row_spec(W), _row_spec(W),
                  _row_spec(W), _row_spec(W),
                  _row_spec(D_EMB), _row_spec(D_EMB),
                  _full_spec((D_EMB, H)), _full_spec((D_EMB, H)),
                  _full_spec((D_EMB, D_EMB)), _full_spec((1, D_EMB)),
                  _full_spec((D_EMB, D_EMB)), _full_spec((1, D_EMB))],
        out_specs=[_row_spec(D_EMB), _row_spec(D_EMB)],
        out_shape=[jax.ShapeDtypeStruct((N_PAD, D_EMB), _f32)] * 2,
    )(p1_item, p1_user, p_item, p_user, ti, tu,
      W_l1_ui, W_l1_iu, W_lin_item, b2(b_lin_item),
      W_lin_user, b2(b_lin_user))

    return (ou[:N], oi[:N])


# speedup vs baseline: 4.1404x; 1.1665x over previous
import functools

import jax
import jax.numpy as jnp
from jax import lax
from jax.experimental import pallas as pl
from jax.experimental.pallas import tpu as pltpu
from jax.experimental.pallas import tpu_sc as plsc

N = 10000
E = 320000
D_IN = 128
H = 64
D_EMB = 128

NC = 1
NS = 16
NW = NC * NS
CHUNK = 128
G = 2
WCH = 16
W = 128
EPT = -(-E // NW)
NCH = -(-(-(-EPT // CHUNK)) // WCH) * WCH
EPT_PAD = NCH * CHUNK
E_PAD = EPT_PAD * NW
N_PAD = 10240
RPS = N_PAD // NS

GRID = 16
BR = N_PAD // GRID

_f32 = jnp.float32


def _dot_t(a, b):
    return lax.dot_general(a, b, (((1,), (1,)), ((), ())),
                           preferred_element_type=_f32)


def _tc1_body(xu_ref, xi_ref, wui_ref, wiu_ref, yu_ref, yi_ref):
    marker = jnp.where(
        lax.broadcasted_iota(jnp.int32, (BR, W - H), 1) == 0, 1.0, 0.0
    ).astype(_f32)
    yu_ref[...] = jnp.concatenate([_dot_t(xu_ref[...], wui_ref[...]), marker], 1)
    yi_ref[...] = jnp.concatenate([_dot_t(xi_ref[...], wiu_ref[...]), marker], 1)


def _tc2_half(p, x, wr0, bl0, wr1, bl1):
    s = p[:, :H]
    cnt = p[:, H:H + 1]
    r = _dot_t(x, wr0) + bl0
    h = jnp.maximum(s / jnp.maximum(cnt, 1.0) + r, 0.0)
    t = _dot_t(h, wr1) + bl1
    hw = jnp.concatenate([h, jnp.zeros((h.shape[0], W - H), _f32)], 1)
    return hw, t


def _tc2_body(pi_ref, pu_ref, xi_ref, xu_ref,
              wr0ui_ref, bl0ui_ref, wr0iu_ref, bl0iu_ref,
              wr1ui_ref, bl1ui_ref, wr1iu_ref, bl1iu_ref,
              hi_ref, hu_ref, ti_ref, tu_ref):
    hi_ref[...], ti_ref[...] = _tc2_half(
        pi_ref[...], xi_ref[...], wr0ui_ref[...], bl0ui_ref[...],
        wr1ui_ref[...], bl1ui_ref[...])
    hu_ref[...], tu_ref[...] = _tc2_half(
        pu_ref[...], xu_ref[...], wr0iu_ref[...], bl0iu_ref[...],
        wr1iu_ref[...], bl1iu_ref[...])


def _tc3_half(p1, p0, t, wl1, wlin, blin):
    s1 = p1[:, :H]
    cnt = p0[:, H:H + 1]
    z = _dot_t(s1 / jnp.maximum(cnt, 1.0), wl1) + t
    return _dot_t(z, wlin) + blin


def _tc3_body(p1i_ref, p1u_ref, p0i_ref, p0u_ref, ti_ref, tu_ref,
              wl1ui_ref, wl1iu_ref, wlini_ref, blini_ref,
              wlinu_ref, blinu_ref, oi_ref, ou_ref):
    oi_ref[...] = _tc3_half(p1i_ref[...], p0i_ref[...], ti_ref[...],
                            wl1ui_ref[...], wlini_ref[...], blini_ref[...])
    ou_ref[...] = _tc3_half(p1u_ref[...], p0u_ref[...], tu_ref[...],
                            wl1iu_ref[...], wlinu_ref[...], blinu_ref[...])


def _row_spec(w):
    return pl.BlockSpec((BR, w), lambda i: (i, 0))


def _full_spec(shape):
    nd = len(shape)
    return pl.BlockSpec(shape, lambda i, _n=nd: (0,) * _n)


def _make_seg_sum():
    mesh = plsc.VectorSubcoreMesh(core_axis_name="c", subcore_axis_name="s",
                                  num_cores=NC, num_subcores=NS)
    NWIN = NCH // WCH
    NB = WCH // G

    @functools.partial(
        pl.kernel,
        out_type=jax.ShapeDtypeStruct((N_PAD, W), _f32),
        mesh=mesh,
        scratch_types=(
            pltpu.VMEM((WCH, CHUNK), jnp.int32),
            pltpu.VMEM((WCH, CHUNK), jnp.int32),
            [pltpu.VMEM((CHUNK, W), _f32) for _ in range(G)],
            pltpu.VMEM_SHARED((N_PAD, W), _f32),
            pltpu.SemaphoreType.DMA,
            pltpu.SemaphoreType.DMA,
        ),
    )
    def seg(tab, src3, dst3, out, srci, dsti, bufs, acc, gsem, ssem):
        sid = lax.axis_index("s")
        wid = sid

        def _zrow(i, carry):
            for c in range(W // 16):
                bufs[0][i, pl.ds(c * 16, 16)] = jnp.zeros((16,), _f32)
            return carry
        lax.fori_loop(0, CHUNK, _zrow, 0)
        for r in range(RPS // CHUNK):
            off = sid * RPS + r * CHUNK
            pltpu.sync_copy(bufs[0], acc.at[pl.ds(off, CHUNK)])
        plsc.subcore_barrier()

        def win(v, carry):
            pltpu.sync_copy(src3.at[wid, pl.ds(v * WCH, WCH)], srci)
            pltpu.sync_copy(dst3.at[wid, pl.ds(v * WCH, WCH)], dsti)

            def grp(g, c2):
                b = g * G
                ghs = [pltpu.async_copy(tab.at[srci.at[b + q]], bufs[q], gsem)
                       for q in range(G)]
                for h in ghs:
                    h.wait()
                shs = [pltpu.async_copy(bufs[q], acc.at[dsti.at[b + q]],
                                        ssem, add=True)
                       for q in range(G)]
                for h in shs:
                    h.wait()
                return c2
            lax.fori_loop(0, NB, grp, 0)
            return carry
        lax.fori_loop(0, NWIN, win, 0)
        plsc.subcore_barrier()

        for r in range(RPS // CHUNK):
            off = sid * RPS + r * CHUNK
            pltpu.sync_copy(acc.at[pl.ds(off, CHUNK)],
                            out.at[pl.ds(off, CHUNK)])

    return seg


_seg_sum_cache = {}


def _seg_sum(tab, src3, dst3):
    if "k" not in _seg_sum_cache:
        _seg_sum_cache["k"] = _make_seg_sum()
    return _seg_sum_cache["k"](tab, src3, dst3)


def _prep_edges(ei):
    src = ei[0].astype(jnp.int32)
    dst = ei[1].astype(jnp.int32)
    npad = E_PAD - E
    padv = N + (jnp.arange(npad, dtype=jnp.int32) % (N_PAD - N))
    src_p = jnp.concatenate([src, padv]).reshape(NW, NCH, CHUNK)
    dst_p = jnp.concatenate([dst, padv]).reshape(NW, NCH, CHUNK)
    return src_p, dst_p


def _pad_rows(x):
    return jnp.pad(x, ((0, N_PAD - N), (0, 0)))


@jax.jit
def kernel(x_user, x_item, edge_index_ui, edge_index_iu,
           W_l0_ui, b_l0_ui, W_r0_ui, W_l0_iu, b_l0_iu, W_r0_iu,
           W_l1_ui, b_l1_ui, W_r1_ui, W_l1_iu, b_l1_iu, W_r1_iu,
           W_lin_user, b_lin_user, W_lin_item, b_lin_item):
    src_ui, dst_ui = _prep_edges(edge_index_ui)
    src_iu, dst_iu = _prep_edges(edge_index_iu)
    xu = _pad_rows(x_user)
    xi = _pad_rows(x_item)
    b2 = lambda b: b.reshape(1, -1)

    yu, yi = pl.pallas_call(
        _tc1_body,
        grid=(GRID,),
        in_specs=[_row_spec(D_IN), _row_spec(D_IN),
                  _full_spec((H, D_IN)), _full_spec((H, D_IN))],
        out_specs=[_row_spec(W), _row_spec(W)],
        out_shape=[jax.ShapeDtypeStruct((N_PAD, W), _f32)] * 2,
    )(xu, xi, W_l0_ui, W_l0_iu)

    p_item = _seg_sum(yu, src_ui, dst_ui)
    p_user = _seg_sum(yi, src_iu, dst_iu)

    hi, hu, ti, tu = pl.pallas_call(
        _tc2_body,
        grid=(GRID,),
        in_specs=[_row_spec(W), _row_spec(W),
                  _row_spec(D_IN), _row_spec(D_IN),
                  _full_spec((H, D_IN)), _full_spec((1, H)),
                  _full_spec((H, D_IN)), _full_spec((1, H)),
                  _full_spec((D_EMB, H)), _full_spec((1, D_EMB)),
                  _full_spec((D_EMB, H)), _full_spec((1, D_EMB))],
        out_specs=[_row_spec(W), _row_spec(W),
                   _row_spec(D_EMB), _row_spec(D_EMB)],
        out_shape=[jax.ShapeDtypeStruct((N_PAD, W), _f32)] * 2 +
                  [jax.ShapeDtypeStruct((N_PAD, D_EMB), _f32)] * 2,
    )(p_item, p_user, xi, xu,
      W_r0_ui, b2(b_l0_ui), W_r0_iu, b2(b_l0_iu),
      W_r1_ui, b2(b_l1_ui), W_r1_iu, b2(b_l1_iu))

    p1_item = _seg_sum(hu, src_ui, dst_ui)
    p1_user = _seg_sum(hi, src_iu, dst_iu)

    oi, ou = pl.pallas_call(
        _tc3_body,
        grid=(GRID,),
        in_specs=[_row_spec(W), _row_spec(W),
                  _row_spec(W), _row_spec(W),
                  _row_spec(D_EMB), _row_spec(D_EMB),
                  _full_spec((D_EMB, H)), _full_spec((D_EMB, H)),
                  _full_spec((D_EMB, D_EMB)), _full_spec((1, D_EMB)),
                  _full_spec((D_EMB, D_EMB)), _full_spec((1, D_EMB))],
        out_specs=[_row_spec(D_EMB), _row_spec(D_EMB)],
        out_shape=[jax.ShapeDtypeStruct((N_PAD, D_EMB), _f32)] * 2,
    )(p1_item, p1_user, p_item, p_user, ti, tu,
      W_l1_ui, W_l1_iu, W_lin_item, b2(b_lin_item),
      W_lin_user, b2(b_lin_user))

    return (ou[:N], oi[:N])


# speedup vs baseline: 7.1419x; 1.7249x over previous
import functools

import jax
import jax.numpy as jnp
from jax import lax
from jax.experimental import pallas as pl
from jax.experimental.pallas import tpu as pltpu
from jax.experimental.pallas import tpu_sc as plsc

N = 10000
E = 320000
D_IN = 128
H = 64
D_EMB = 128

NC = 2
NS = 16
NW = NC * NS
CHUNK = 128
G = 2
WCH = 16
W = 128
EPT = -(-E // NW)
NCH = -(-(-(-EPT // CHUNK)) // WCH) * WCH
EPT_PAD = NCH * CHUNK
E_PAD = EPT_PAD * NW
N_PAD = 10240
RPS = N_PAD // NS

GRID = 16
BR = N_PAD // GRID

_f32 = jnp.float32


def _dot_t(a, b):
    return lax.dot_general(a, b, (((1,), (1,)), ((), ())),
                           preferred_element_type=_f32)


def _tc1_body(xu_ref, xi_ref, wui_ref, wiu_ref, yu_ref, yi_ref):
    marker = jnp.where(
        lax.broadcasted_iota(jnp.int32, (BR, W - H), 1) == 0, 1.0, 0.0
    ).astype(_f32)
    yu_ref[...] = jnp.concatenate([_dot_t(xu_ref[...], wui_ref[...]), marker], 1)
    yi_ref[...] = jnp.concatenate([_dot_t(xi_ref[...], wiu_ref[...]), marker], 1)


def _tc2_half(p, x, wr0, bl0, wr1, bl1):
    s = p[0, :, :H] + p[1, :, :H]
    cnt = p[0, :, H:H + 1] + p[1, :, H:H + 1]
    r = _dot_t(x, wr0) + bl0
    h = jnp.maximum(s / jnp.maximum(cnt, 1.0) + r, 0.0)
    t = _dot_t(h, wr1) + bl1
    hw = jnp.concatenate([h, jnp.zeros((h.shape[0], W - H), _f32)], 1)
    return hw, t


def _tc2_body(pi_ref, pu_ref, xi_ref, xu_ref,
              wr0ui_ref, bl0ui_ref, wr0iu_ref, bl0iu_ref,
              wr1ui_ref, bl1ui_ref, wr1iu_ref, bl1iu_ref,
              hi_ref, hu_ref, ti_ref, tu_ref):
    hi_ref[...], ti_ref[...] = _tc2_half(
        pi_ref[...], xi_ref[...], wr0ui_ref[...], bl0ui_ref[...],
        wr1ui_ref[...], bl1ui_ref[...])
    hu_ref[...], tu_ref[...] = _tc2_half(
        pu_ref[...], xu_ref[...], wr0iu_ref[...], bl0iu_ref[...],
        wr1iu_ref[...], bl1iu_ref[...])


def _tc3_half(p1, p0, t, wl1, wlin, blin):
    s1 = p1[0, :, :H] + p1[1, :, :H]
    cnt = p0[0, :, H:H + 1] + p0[1, :, H:H + 1]
    z = _dot_t(s1 / jnp.maximum(cnt, 1.0), wl1) + t
    return _dot_t(z, wlin) + blin


def _tc3_body(p1i_ref, p1u_ref, p0i_ref, p0u_ref, ti_ref, tu_ref,
              wl1ui_ref, wl1iu_ref, wlini_ref, blini_ref,
              wlinu_ref, blinu_ref, oi_ref, ou_ref):
    oi_ref[...] = _tc3_half(p1i_ref[...], p0i_ref[...], ti_ref[...],
                            wl1ui_ref[...], wlini_ref[...], blini_ref[...])
    ou_ref[...] = _tc3_half(p1u_ref[...], p0u_ref[...], tu_ref[...],
                            wl1iu_ref[...], wlinu_ref[...], blinu_ref[...])


def _row_spec(w):
    return pl.BlockSpec((BR, w), lambda i: (i, 0))


def _part_spec(w):
    return pl.BlockSpec((2, BR, w), lambda i: (0, i, 0))


def _full_spec(shape):
    nd = len(shape)
    return pl.BlockSpec(shape, lambda i, _n=nd: (0,) * _n)


def _make_seg_sum():
    mesh = plsc.VectorSubcoreMesh(core_axis_name="c", subcore_axis_name="s",
                                  num_cores=NC, num_subcores=NS)
    NWIN = NCH // WCH
    NB = WCH // G

    @functools.partial(
        pl.kernel,
        out_type=jax.ShapeDtypeStruct((NC * N_PAD, W), _f32),
        mesh=mesh,
        scratch_types=(
            pltpu.VMEM((WCH, CHUNK), jnp.int32),
            pltpu.VMEM((WCH, CHUNK), jnp.int32),
            [pltpu.VMEM((CHUNK, W), _f32) for _ in range(G)],
            pltpu.VMEM_SHARED((N_PAD, W), _f32),
            pltpu.SemaphoreType.DMA,
            pltpu.SemaphoreType.DMA,
        ),
    )
    def seg(tab, src3, dst3, out, srci, dsti, bufs, acc, gsem, ssem):
        cid = lax.axis_index("c")
        sid = lax.axis_index("s")
        wid = sid * NC + cid

        def _zrow(i, carry):
            for c in range(W // 16):
                bufs[0][i, pl.ds(c * 16, 16)] = jnp.zeros((16,), _f32)
            return carry
        lax.fori_loop(0, CHUNK, _zrow, 0)
        for r in range(RPS // CHUNK):
            off = sid * RPS + r * CHUNK
            pltpu.sync_copy(bufs[0], acc.at[pl.ds(off, CHUNK)])
        plsc.subcore_barrier()

        def win(v, carry):
            pltpu.sync_copy(src3.at[wid, pl.ds(v * WCH, WCH)], srci)
            pltpu.sync_copy(dst3.at[wid, pl.ds(v * WCH, WCH)], dsti)

            def grp(g, c2):
                b = g * G
                ghs = [pltpu.async_copy(tab.at[srci.at[b + q]], bufs[q], gsem)
                       for q in range(G)]
                for h in ghs:
                    h.wait()
                shs = [pltpu.async_copy(bufs[q], acc.at[dsti.at[b + q]],
                                        ssem, add=True)
                       for q in range(G)]
                for h in shs:
                    h.wait()
                return c2
            lax.fori_loop(0, NB, grp, 0)
            return carry
        lax.fori_loop(0, NWIN, win, 0)
        plsc.subcore_barrier()

        for r in range(RPS // CHUNK):
            off = sid * RPS + r * CHUNK
            pltpu.sync_copy(acc.at[pl.ds(off, CHUNK)],
                            out.at[pl.ds(cid * N_PAD + off, CHUNK)])

    return seg


_seg_sum_cache = {}


def _seg_sum(tab, src3, dst3):
    if "k" not in _seg_sum_cache:
        _seg_sum_cache["k"] = _make_seg_sum()
    return _seg_sum_cache["k"](tab, src3, dst3).reshape(NC, N_PAD, W)


def _prep_edges(ei):
    src = ei[0].astype(jnp.int32)
    dst = ei[1].astype(jnp.int32)
    npad = E_PAD - E
    padv = N + (jnp.arange(npad, dtype=jnp.int32) % (N_PAD - N))
    src_p = jnp.concatenate([src, padv]).reshape(NW, NCH, CHUNK)
    dst_p = jnp.concatenate([dst, padv]).reshape(NW, NCH, CHUNK)
    return src_p, dst_p


def _pad_rows(x):
    return jnp.pad(x, ((0, N_PAD - N), (0, 0)))


@jax.jit
def kernel(x_user, x_item, edge_index_ui, edge_index_iu,
           W_l0_ui, b_l0_ui, W_r0_ui, W_l0_iu, b_l0_iu, W_r0_iu,
           W_l1_ui, b_l1_ui, W_r1_ui, W_l1_iu, b_l1_iu, W_r1_iu,
           W_lin_user, b_lin_user, W_lin_item, b_lin_item):
    src_ui, dst_ui = _prep_edges(edge_index_ui)
    src_iu, dst_iu = _prep_edges(edge_index_iu)
    xu = _pad_rows(x_user)
    xi = _pad_rows(x_item)
    b2 = lambda b: b.reshape(1, -1)

    yu, yi = pl.pallas_call(
        _tc1_body,
        grid=(GRID,),
        in_specs=[_row_spec(D_IN), _row_spec(D_IN),
                  _full_spec((H, D_IN)), _full_spec((H, D_IN))],
        out_specs=[_row_spec(W), _row_spec(W)],
        out_shape=[jax.ShapeDtypeStruct((N_PAD, W), _f32)] * 2,
    )(xu, xi, W_l0_ui, W_l0_iu)

    p_item = _seg_sum(yu, src_ui, dst_ui)
    p_user = _seg_sum(yi, src_iu, dst_iu)

    hi, hu, ti, tu = pl.pallas_call(
        _tc2_body,
        grid=(GRID,),
        in_specs=[_part_spec(W), _part_spec(W),
                  _row_spec(D_IN), _row_spec(D_IN),
                  _full_spec((H, D_IN)), _full_spec((1, H)),
                  _full_spec((H, D_IN)), _full_spec((1, H)),
                  _full_spec((D_EMB, H)), _full_spec((1, D_EMB)),
                  _full_spec((D_EMB, H)), _full_spec((1, D_EMB))],
        out_specs=[_row_spec(W), _row_spec(W),
                   _row_spec(D_EMB), _row_spec(D_EMB)],
        out_shape=[jax.ShapeDtypeStruct((N_PAD, W), _f32)] * 2 +
                  [jax.ShapeDtypeStruct((N_PAD, D_EMB), _f32)] * 2,
    )(p_item, p_user, xi, xu,
      W_r0_ui, b2(b_l0_ui), W_r0_iu, b2(b_l0_iu),
      W_r1_ui, b2(b_l1_ui), W_r1_iu, b2(b_l1_iu))

    p1_item = _seg_sum(hu, src_ui, dst_ui)
    p1_user = _seg_sum(hi, src_iu, dst_iu)

    oi, ou = pl.pallas_call(
        _tc3_body,
        grid=(GRID,),
        in_specs=[_part_spec(W), _part_spec(W),
                  _part_spec(W), _part_spec(W),
                  _row_spec(D_EMB), _row_spec(D_EMB),
                  _full_spec((D_EMB, H)), _full_spec((D_EMB, H)),
                  _full_spec((D_EMB, D_EMB)), _full_spec((1, D_EMB)),
                  _full_spec((D_EMB, D_EMB)), _full_spec((1, D_EMB))],
        out_specs=[_row_spec(D_EMB), _row_spec(D_EMB)],
        out_shape=[jax.ShapeDtypeStruct((N_PAD, D_EMB), _f32)] * 2,
    )(p1_item, p1_user, p_item, p_user, ti, tu,
      W_l1_ui, W_l1_iu, W_lin_item, b2(b_lin_item),
      W_lin_user, b2(b_lin_user))

    return (ou[:N], oi[:N])


# speedup vs baseline: 9.1831x; 1.2858x over previous
import functools

import jax
import jax.numpy as jnp
from jax import lax
from jax.experimental import pallas as pl
from jax.experimental.pallas import tpu as pltpu
from jax.experimental.pallas import tpu_sc as plsc

N = 10000
E = 320000
D_IN = 128
H = 64
D_EMB = 128

NC = 2
NS = 16
NW = NC * NS
CHUNK = 128
G = 2
WCH = 16
W = 128
EPT = -(-E // NW)
NCH = -(-(-(-EPT // CHUNK)) // WCH) * WCH
EPT_PAD = NCH * CHUNK
E_PAD = EPT_PAD * NW
N_PAD = 10240
RPS = N_PAD // NS

GRID = 16
BR = N_PAD // GRID

_f32 = jnp.float32


def _dot_t(a, b):
    return lax.dot_general(a, b, (((1,), (1,)), ((), ())),
                           preferred_element_type=_f32)


def _tc1_body(xu_ref, xi_ref, wui_ref, wiu_ref, yu_ref, yi_ref):
    marker = jnp.where(
        lax.broadcasted_iota(jnp.int32, (BR, W - H), 1) == 0, 1.0, 0.0
    ).astype(_f32)
    yu_ref[...] = jnp.concatenate([_dot_t(xu_ref[...], wui_ref[...]), marker], 1)
    yi_ref[...] = jnp.concatenate([_dot_t(xi_ref[...], wiu_ref[...]), marker], 1)


def _tc2_half(p, x, wr0, bl0, wr1, bl1):
    s = p[0, :, :H] + p[1, :, :H]
    cnt = p[0, :, H:H + 1] + p[1, :, H:H + 1]
    r = _dot_t(x, wr0) + bl0
    h = jnp.maximum(s / jnp.maximum(cnt, 1.0) + r, 0.0)
    t = _dot_t(h, wr1) + bl1
    hw = jnp.concatenate([h, jnp.zeros((h.shape[0], W - H), _f32)], 1)
    return hw, t


def _tc2_body(pi_ref, pu_ref, xi_ref, xu_ref,
              wr0ui_ref, bl0ui_ref, wr0iu_ref, bl0iu_ref,
              wr1ui_ref, bl1ui_ref, wr1iu_ref, bl1iu_ref,
              hi_ref, hu_ref, ti_ref, tu_ref):
    hi_ref[...], ti_ref[...] = _tc2_half(
        pi_ref[...], xi_ref[...], wr0ui_ref[...], bl0ui_ref[...],
        wr1ui_ref[...], bl1ui_ref[...])
    hu_ref[...], tu_ref[...] = _tc2_half(
        pu_ref[...], xu_ref[...], wr0iu_ref[...], bl0iu_ref[...],
        wr1iu_ref[...], bl1iu_ref[...])


def _tc3_half(p1, p0, t, wl1, wlin, blin):
    s1 = p1[0, :, :H] + p1[1, :, :H]
    cnt = p0[0, :, H:H + 1] + p0[1, :, H:H + 1]
    z = _dot_t(s1 / jnp.maximum(cnt, 1.0), wl1) + t
    return _dot_t(z, wlin) + blin


def _tc3_body(p1i_ref, p1u_ref, p0i_ref, p0u_ref, ti_ref, tu_ref,
              wl1ui_ref, wl1iu_ref, wlini_ref, blini_ref,
              wlinu_ref, blinu_ref, oi_ref, ou_ref):
    oi_ref[...] = _tc3_half(p1i_ref[...], p0i_ref[...], ti_ref[...],
                            wl1ui_ref[...], wlini_ref[...], blini_ref[...])
    ou_ref[...] = _tc3_half(p1u_ref[...], p0u_ref[...], tu_ref[...],
                            wl1iu_ref[...], wlinu_ref[...], blinu_ref[...])


def _row_spec(w):
    return pl.BlockSpec((BR, w), lambda i: (i, 0))


def _part_spec(w):
    return pl.BlockSpec((2, BR, w), lambda i: (0, i, 0))


def _full_spec(shape):
    nd = len(shape)
    return pl.BlockSpec(shape, lambda i, _n=nd: (0,) * _n)


def _make_seg_sum():
    mesh = plsc.VectorSubcoreMesh(core_axis_name="c", subcore_axis_name="s",
                                  num_cores=NC, num_subcores=NS)
    NWIN = NCH // WCH
    NB = WCH // G

    @functools.partial(
        pl.kernel,
        out_type=jax.ShapeDtypeStruct((NC * N_PAD, W), _f32),
        mesh=mesh,
        scratch_types=(
            pltpu.VMEM((WCH, CHUNK), jnp.int32),
            pltpu.VMEM((WCH, CHUNK), jnp.int32),
            [pltpu.VMEM((CHUNK, W), _f32) for _ in range(G)],
            pltpu.VMEM_SHARED((N_PAD, W), _f32),
            pltpu.SemaphoreType.DMA,
            pltpu.SemaphoreType.DMA,
        ),
    )
    def seg(tab, src3, dst3, out, srci, dsti, bufs, acc, gsem, ssem):
        cid = lax.axis_index("c")
        sid = lax.axis_index("s")
        wid = sid * NC + cid

        def _zrow(i, carry):
            for c in range(W // 16):
                bufs[0][i, pl.ds(c * 16, 16)] = jnp.zeros((16,), _f32)
            return carry
        lax.fori_loop(0, CHUNK, _zrow, 0)
        for r in range(RPS // CHUNK):
            off = sid * RPS + r * CHUNK
            pltpu.sync_copy(bufs[0], acc.at[pl.ds(off, CHUNK)])
        plsc.subcore_barrier()

        def win(v, carry):
            pltpu.sync_copy(src3.at[wid, pl.ds(v * WCH, WCH)], srci)
            pltpu.sync_copy(dst3.at[wid, pl.ds(v * WCH, WCH)], dsti)

            ghs = {
                0: pltpu.async_copy(tab.at[srci.at[0]], bufs[0], gsem),
                1: pltpu.async_copy(tab.at[srci.at[1]], bufs[1], gsem),
            }
            shs = {}
            for c in range(WCH):
                ghs[c].wait()
                shs[c] = pltpu.async_copy(bufs[c % G], acc.at[dsti.at[c]],
                                          ssem, add=True)
                if c + G < WCH:
                    shs[c].wait()
                    ghs[c + G] = pltpu.async_copy(tab.at[srci.at[c + G]],
                                                  bufs[c % G], gsem)
            for c in range(WCH - G, WCH):
                shs[c].wait()
            return carry
        lax.fori_loop(0, NWIN, win, 0)
        plsc.subcore_barrier()

        for r in range(RPS // CHUNK):
            off = sid * RPS + r * CHUNK
            pltpu.sync_copy(acc.at[pl.ds(off, CHUNK)],
                            out.at[pl.ds(cid * N_PAD + off, CHUNK)])

    return seg


_seg_sum_cache = {}


def _seg_sum(tab, src3, dst3):
    if "k" not in _seg_sum_cache:
        _seg_sum_cache["k"] = _make_seg_sum()
    return _seg_sum_cache["k"](tab, src3, dst3).reshape(NC, N_PAD, W)


def _prep_edges(ei):
    src = ei[0].astype(jnp.int32)
    dst = ei[1].astype(jnp.int32)
    npad = E_PAD - E
    padv = N + (jnp.arange(npad, dtype=jnp.int32) % (N_PAD - N))
    src_p = jnp.concatenate([src, padv]).reshape(NW, NCH, CHUNK)
    dst_p = jnp.concatenate([dst, padv]).reshape(NW, NCH, CHUNK)
    return src_p, dst_p


def _pad_rows(x):
    return jnp.pad(x, ((0, N_PAD - N), (0, 0)))


@jax.jit
def kernel(x_user, x_item, edge_index_ui, edge_index_iu,
           W_l0_ui, b_l0_ui, W_r0_ui, W_l0_iu, b_l0_iu, W_r0_iu,
           W_l1_ui, b_l1_ui, W_r1_ui, W_l1_iu, b_l1_iu, W_r1_iu,
           W_lin_user, b_lin_user, W_lin_item, b_lin_item):
    src_ui, dst_ui = _prep_edges(edge_index_ui)
    src_iu, dst_iu = _prep_edges(edge_index_iu)
    xu = _pad_rows(x_user)
    xi = _pad_rows(x_item)
    b2 = lambda b: b.reshape(1, -1)

    yu, yi = pl.pallas_call(
        _tc1_body,
        grid=(GRID,),
        in_specs=[_row_spec(D_IN), _row_spec(D_IN),
                  _full_spec((H, D_IN)), _full_spec((H, D_IN))],
        out_specs=[_row_spec(W), _row_spec(W)],
        out_shape=[jax.ShapeDtypeStruct((N_PAD, W), _f32)] * 2,
    )(xu, xi, W_l0_ui, W_l0_iu)

    p_item = _seg_sum(yu, src_ui, dst_ui)
    p_user = _seg_sum(yi, src_iu, dst_iu)

    hi, hu, ti, tu = pl.pallas_call(
        _tc2_body,
        grid=(GRID,),
        in_specs=[_part_spec(W), _part_spec(W),
                  _row_spec(D_IN), _row_spec(D_IN),
                  _full_spec((H, D_IN)), _full_spec((1, H)),
                  _full_spec((H, D_IN)), _full_spec((1, H)),
                  _full_spec((D_EMB, H)), _full_spec((1, D_EMB)),
                  _full_spec((D_EMB, H)), _full_spec((1, D_EMB))],
        out_specs=[_row_spec(W), _row_spec(W),
                   _row_spec(D_EMB), _row_spec(D_EMB)],
        out_shape=[jax.ShapeDtypeStruct((N_PAD, W), _f32)] * 2 +
                  [jax.ShapeDtypeStruct((N_PAD, D_EMB), _f32)] * 2,
    )(p_item, p_user, xi, xu,
      W_r0_ui, b2(b_l0_ui), W_r0_iu, b2(b_l0_iu),
      W_r1_ui, b2(b_l1_ui), W_r1_iu, b2(b_l1_iu))

    p1_item = _seg_sum(hu, src_ui, dst_ui)
    p1_user = _seg_sum(hi, src_iu, dst_iu)

    oi, ou = pl.pallas_call(
        _tc3_body,
        grid=(GRID,),
        in_specs=[_part_spec(W), _part_spec(W),
                  _part_spec(W), _part_spec(W),
                  _row_spec(D_EMB), _row_spec(D_EMB),
                  _full_spec((D_EMB, H)), _full_spec((D_EMB, H)),
                  _full_spec((D_EMB, D_EMB)), _full_spec((1, D_EMB)),
                  _full_spec((D_EMB, D_EMB)), _full_spec((1, D_EMB))],
        out_specs=[_row_spec(D_EMB), _row_spec(D_EMB)],
        out_shape=[jax.ShapeDtypeStruct((N_PAD, D_EMB), _f32)] * 2,
    )(p1_item, p1_user, p_item, p_user, ti, tu,
      W_l1_ui, W_l1_iu, W_lin_item, b2(b_lin_item),
      W_lin_user, b2(b_lin_user))

    return (ou[:N], oi[:N])


# speedup vs baseline: 9.7046x; 1.0568x over previous
import functools

import jax
import jax.numpy as jnp
from jax import lax
from jax.experimental import pallas as pl
from jax.experimental.pallas import tpu as pltpu
from jax.experimental.pallas import tpu_sc as plsc

N = 10000
E = 320000
D_IN = 128
H = 64
D_EMB = 128

NC = 2
NS = 16
NW = NC * NS
CHUNK = 128
G = 2
WCH = 16
W = 128
EPT = -(-E // NW)
NCH = -(-(-(-EPT // CHUNK)) // WCH) * WCH
EPT_PAD = NCH * CHUNK
E_PAD = EPT_PAD * NW
N_PAD = 10240
RPS = N_PAD // NS

GRID = 16
BR = N_PAD // GRID

_f32 = jnp.float32


def _dot_t(a, b):
    return lax.dot_general(a, b, (((1,), (1,)), ((), ())),
                           preferred_element_type=_f32)


def _tc1_body(xu_ref, xi_ref, wui_ref, wiu_ref, yu_ref, yi_ref):
    marker = jnp.where(
        lax.broadcasted_iota(jnp.int32, (BR, W - H), 1) == 0, 1.0, 0.0
    ).astype(_f32)
    yu_ref[...] = jnp.concatenate([_dot_t(xu_ref[...], wui_ref[...]), marker], 1)
    yi_ref[...] = jnp.concatenate([_dot_t(xi_ref[...], wiu_ref[...]), marker], 1)


def _tc2_half(p, x, wr0, bl0, wr1, bl1):
    s = p[0, :, :H] + p[1, :, :H]
    cnt = p[0, :, H:H + 1] + p[1, :, H:H + 1]
    r = _dot_t(x, wr0) + bl0
    h = jnp.maximum(s / jnp.maximum(cnt, 1.0) + r, 0.0)
    t = _dot_t(h, wr1) + bl1
    hw = jnp.concatenate([h, jnp.zeros((h.shape[0], W - H), _f32)], 1)
    return hw, t


def _tc2_body(pi_ref, pu_ref, xi_ref, xu_ref,
              wr0ui_ref, bl0ui_ref, wr0iu_ref, bl0iu_ref,
              wr1ui_ref, bl1ui_ref, wr1iu_ref, bl1iu_ref,
              hi_ref, hu_ref, ti_ref, tu_ref):
    hi_ref[...], ti_ref[...] = _tc2_half(
        pi_ref[...], xi_ref[...], wr0ui_ref[...], bl0ui_ref[...],
        wr1ui_ref[...], bl1ui_ref[...])
    hu_ref[...], tu_ref[...] = _tc2_half(
        pu_ref[...], xu_ref[...], wr0iu_ref[...], bl0iu_ref[...],
        wr1iu_ref[...], bl1iu_ref[...])


def _tc3_half(p1, p0, t, wl1, wlin, blin):
    s1 = p1[0, :, :H] + p1[1, :, :H]
    cnt = p0[0, :, H:H + 1] + p0[1, :, H:H + 1]
    z = _dot_t(s1 / jnp.maximum(cnt, 1.0), wl1) + t
    return _dot_t(z, wlin) + blin


def _tc3_body(p1i_ref, p1u_ref, p0i_ref, p0u_ref, ti_ref, tu_ref,
              wl1ui_ref, wl1iu_ref, wlini_ref, blini_ref,
              wlinu_ref, blinu_ref, oi_ref, ou_ref):
    oi_ref[...] = _tc3_half(p1i_ref[...], p0i_ref[...], ti_ref[...],
                            wl1ui_ref[...], wlini_ref[...], blini_ref[...])
    ou_ref[...] = _tc3_half(p1u_ref[...], p0u_ref[...], tu_ref[...],
                            wl1iu_ref[...], wlinu_ref[...], blinu_ref[...])


def _row_spec(w):
    return pl.BlockSpec((BR, w), lambda i: (i, 0))


def _part_spec(w):
    return pl.BlockSpec((2, BR, w), lambda i: (0, i, 0))


def _full_spec(shape):
    nd = len(shape)
    return pl.BlockSpec(shape, lambda i, _n=nd: (0,) * _n)


def _make_seg_sum():
    mesh = plsc.VectorSubcoreMesh(core_axis_name="c", subcore_axis_name="s",
                                  num_cores=NC, num_subcores=NS)
    NWIN = NCH // WCH
    NB = WCH // G

    @functools.partial(
        pl.kernel,
        out_type=jax.ShapeDtypeStruct((NC * N_PAD, W), _f32),
        mesh=mesh,
        scratch_types=(
            pltpu.VMEM((2 * WCH, CHUNK), jnp.int32),
            pltpu.VMEM((2 * WCH, CHUNK), jnp.int32),
            [pltpu.VMEM((CHUNK, W), _f32) for _ in range(G)],
            pltpu.VMEM_SHARED((N_PAD, W), _f32),
            pltpu.SemaphoreType.DMA,
            pltpu.SemaphoreType.DMA,
            pltpu.SemaphoreType.DMA,
        ),
    )
    def seg(tab, idx3, out, ibufa, ibufb, bufs, acc, gsem, ssem, isem):
        cid = lax.axis_index("c")
        sid = lax.axis_index("s")
        wid = sid * NC + cid

        def _zrow(i, carry):
            for c in range(W // 16):
                bufs[0][i, pl.ds(c * 16, 16)] = jnp.zeros((16,), _f32)
            return carry
        lax.fori_loop(0, CHUNK, _zrow, 0)
        for r in range(RPS // CHUNK):
            off = sid * RPS + r * CHUNK
            pltpu.sync_copy(bufs[0], acc.at[pl.ds(off, CHUNK)])
        plsc.subcore_barrier()

        ibufs = (ibufa, ibufb)
        pltpu.sync_copy(idx3.at[wid, pl.ds(0, 2 * WCH)], ibufa)
        ihs = {}
        if NWIN > 1:
            ihs[1] = pltpu.async_copy(idx3.at[wid, pl.ds(2 * WCH, 2 * WCH)],
                                      ibufb, isem)
        idx_ready = {0: True}
        pref_done = {}

        def _srow(c, kind):
            v = c // WCH
            if kind == 0 and v not in idx_ready:
                ihs[v].wait()
                idx_ready[v] = True
            if kind == 1 and v not in pref_done:
                pref_done[v] = True
                nv = v + 1
                if nv < NWIN and nv not in ihs:
                    ihs[nv] = pltpu.async_copy(
                        idx3.at[wid, pl.ds(nv * 2 * WCH, 2 * WCH)],
                        ibufs[nv % 2], isem)
            return ibufs[v % 2].at[2 * (c % WCH) + kind]

        ghs = {
            0: pltpu.async_copy(tab.at[_srow(0, 0)], bufs[0], gsem),
            1: pltpu.async_copy(tab.at[_srow(1, 0)], bufs[1], gsem),
        }
        shs = {}
        for c in range(NCH):
            ghs[c].wait()
            shs[c] = pltpu.async_copy(bufs[c % G], acc.at[_srow(c, 1)],
                                      ssem, add=True)
            if c + G < NCH:
                shs[c].wait()
                ghs[c + G] = pltpu.async_copy(tab.at[_srow(c + G, 0)],
                                              bufs[c % G], gsem)
        for c in range(NCH - G, NCH):
            shs[c].wait()
        plsc.subcore_barrier()

        for r in range(RPS // CHUNK):
            off = sid * RPS + r * CHUNK
            pltpu.sync_copy(acc.at[pl.ds(off, CHUNK)],
                            out.at[pl.ds(cid * N_PAD + off, CHUNK)])

    return seg


_seg_sum_cache = {}


def _seg_sum(tab, idx3):
    if "k" not in _seg_sum_cache:
        _seg_sum_cache["k"] = _make_seg_sum()
    return _seg_sum_cache["k"](tab, idx3).reshape(NC, N_PAD, W)


def _prep_edges(ei):
    src = ei[0].astype(jnp.int32)
    dst = ei[1].astype(jnp.int32)
    npad = E_PAD - E
    padv = N + (jnp.arange(npad, dtype=jnp.int32) % (N_PAD - N))
    src_p = jnp.concatenate([src, padv]).reshape(NW, NCH, 1, CHUNK)
    dst_p = jnp.concatenate([dst, padv]).reshape(NW, NCH, 1, CHUNK)
    return jnp.concatenate([src_p, dst_p], 2).reshape(NW, NCH * 2, CHUNK)


def _pad_rows(x):
    return jnp.pad(x, ((0, N_PAD - N), (0, 0)))


@jax.jit
def kernel(x_user, x_item, edge_index_ui, edge_index_iu,
           W_l0_ui, b_l0_ui, W_r0_ui, W_l0_iu, b_l0_iu, W_r0_iu,
           W_l1_ui, b_l1_ui, W_r1_ui, W_l1_iu, b_l1_iu, W_r1_iu,
           W_lin_user, b_lin_user, W_lin_item, b_lin_item):
    idx_ui = _prep_edges(edge_index_ui)
    idx_iu = _prep_edges(edge_index_iu)
    xu = _pad_rows(x_user)
    xi = _pad_rows(x_item)
    b2 = lambda b: b.reshape(1, -1)

    yu, yi = pl.pallas_call(
        _tc1_body,
        grid=(GRID,),
        in_specs=[_row_spec(D_IN), _row_spec(D_IN),
                  _full_spec((H, D_IN)), _full_spec((H, D_IN))],
        out_specs=[_row_spec(W), _row_spec(W)],
        out_shape=[jax.ShapeDtypeStruct((N_PAD, W), _f32)] * 2,
    )(xu, xi, W_l0_ui, W_l0_iu)

    p_item = _seg_sum(yu, idx_ui)
    p_user = _seg_sum(yi, idx_iu)

    hi, hu, ti, tu = pl.pallas_call(
        _tc2_body,
        grid=(GRID,),
        in_specs=[_part_spec(W), _part_spec(W),
                  _row_spec(D_IN), _row_spec(D_IN),
                  _full_spec((H, D_IN)), _full_spec((1, H)),
                  _full_spec((H, D_IN)), _full_spec((1, H)),
                  _full_spec((D_EMB, H)), _full_spec((1, D_EMB)),
                  _full_spec((D_EMB, H)), _full_spec((1, D_EMB))],
        out_specs=[_row_spec(W), _row_spec(W),
                   _row_spec(D_EMB), _row_spec(D_EMB)],
        out_shape=[jax.ShapeDtypeStruct((N_PAD, W), _f32)] * 2 +
                  [jax.ShapeDtypeStruct((N_PAD, D_EMB), _f32)] * 2,
    )(p_item, p_user, xi, xu,
      W_r0_ui, b2(b_l0_ui), W_r0_iu, b2(b_l0_iu),
      W_r1_ui, b2(b_l1_ui), W_r1_iu, b2(b_l1_iu))

    p1_item = _seg_sum(hu, idx_ui)
    p1_user = _seg_sum(hi, idx_iu)

    oi, ou = pl.pallas_call(
        _tc3_body,
        grid=(GRID,),
        in_specs=[_part_spec(W), _part_spec(W),
                  _part_spec(W), _part_spec(W),
                  _row_spec(D_EMB), _row_spec(D_EMB),
                  _full_spec((D_EMB, H)), _full_spec((D_EMB, H)),
                  _full_spec((D_EMB, D_EMB)), _full_spec((1, D_EMB)),
                  _full_spec((D_EMB, D_EMB)), _full_spec((1, D_EMB))],
        out_specs=[_row_spec(D_EMB), _row_spec(D_EMB)],
        out_shape=[jax.ShapeDtypeStruct((N_PAD, D_EMB), _f32)] * 2,
    )(p1_item, p1_user, p_item, p_user, ti, tu,
      W_l1_ui, W_l1_iu, W_lin_item, b2(b_lin_item),
      W_lin_user, b2(b_lin_user))

    return (ou[:N], oi[:N])


# speedup vs baseline: 9.7083x; 1.0004x over previous
import functools

import jax
import jax.numpy as jnp
from jax import lax
from jax.experimental import pallas as pl
from jax.experimental.pallas import tpu as pltpu
from jax.experimental.pallas import tpu_sc as plsc

N = 10000
E = 320000
D_IN = 128
H = 64
D_EMB = 128

NC = 2
NS = 16
NW = NC * NS
CHUNK = 128
G = 2
WCH = 16
W = 128
EPT = -(-E // NW)
NCH = -(-(-(-EPT // CHUNK)) // WCH) * WCH
EPT_PAD = NCH * CHUNK
E_PAD = EPT_PAD * NW
N_PAD = 10240
RPS = N_PAD // NS

GRID = 16
BR = N_PAD // GRID

_f32 = jnp.float32


def _dot_t(a, b):
    return lax.dot_general(a, b, (((1,), (1,)), ((), ())),
                           preferred_element_type=_f32)


def _tc1_body(xu_ref, xi_ref, wui_ref, wiu_ref, yu_ref, yi_ref):
    marker = jnp.where(
        lax.broadcasted_iota(jnp.int32, (BR, W - H), 1) == 0, 1.0, 0.0
    ).astype(_f32)
    yu_ref[...] = jnp.concatenate([_dot_t(xu_ref[...], wui_ref[...]), marker], 1)
    yi_ref[...] = jnp.concatenate([_dot_t(xi_ref[...], wiu_ref[...]), marker], 1)


def _tc2_half(p, x, wr0, bl0, wr1, bl1):
    s = p[0, :, :H] + p[1, :, :H]
    cnt = p[0, :, H:H + 1] + p[1, :, H:H + 1]
    r = _dot_t(x, wr0) + bl0
    h = jnp.maximum(s / jnp.maximum(cnt, 1.0) + r, 0.0)
    t = _dot_t(h, wr1) + bl1
    hw = jnp.concatenate([h, jnp.zeros((h.shape[0], W - H), _f32)], 1)
    return hw, t


def _tc2_body(pi_ref, pu_ref, xi_ref, xu_ref,
              wr0ui_ref, bl0ui_ref, wr0iu_ref, bl0iu_ref,
              wr1ui_ref, bl1ui_ref, wr1iu_ref, bl1iu_ref,
              hi_ref, hu_ref, ti_ref, tu_ref):
    hi_ref[...], ti_ref[...] = _tc2_half(
        pi_ref[...], xi_ref[...], wr0ui_ref[...], bl0ui_ref[...],
        wr1ui_ref[...], bl1ui_ref[...])
    hu_ref[...], tu_ref[...] = _tc2_half(
        pu_ref[...], xu_ref[...], wr0iu_ref[...], bl0iu_ref[...],
        wr1iu_ref[...], bl1iu_ref[...])


def _tc3_half(p1, p0, t, wl1, wlin, blin):
    s1 = p1[0, :, :H] + p1[1, :, :H]
    cnt = p0[0, :, H:H + 1] + p0[1, :, H:H + 1]
    z = _dot_t(s1 / jnp.maximum(cnt, 1.0), wl1) + t
    return _dot_t(z, wlin) + blin


def _tc3_body(p1i_ref, p1u_ref, p0i_ref, p0u_ref, ti_ref, tu_ref,
              wl1ui_ref, wl1iu_ref, wlini_ref, blini_ref,
              wlinu_ref, blinu_ref, oi_ref, ou_ref):
    oi_ref[...] = _tc3_half(p1i_ref[...], p0i_ref[...], ti_ref[...],
                            wl1ui_ref[...], wlini_ref[...], blini_ref[...])
    ou_ref[...] = _tc3_half(p1u_ref[...], p0u_ref[...], tu_ref[...],
                            wl1iu_ref[...], wlinu_ref[...], blinu_ref[...])


def _row_spec(w):
    return pl.BlockSpec((BR, w), lambda i: (i, 0))


def _part_spec(w):
    return pl.BlockSpec((2, BR, w), lambda i: (0, i, 0))


def _full_spec(shape):
    nd = len(shape)
    return pl.BlockSpec(shape, lambda i, _n=nd: (0,) * _n)


def _make_seg_sum():
    mesh = plsc.VectorSubcoreMesh(core_axis_name="c", subcore_axis_name="s",
                                  num_cores=NC, num_subcores=NS)
    NWIN = NCH // WCH

    @functools.partial(
        pl.kernel,
        out_type=jax.ShapeDtypeStruct((NC * N_PAD, W), _f32),
        mesh=mesh,
        scratch_types=(
            pltpu.VMEM((2 * WCH, CHUNK), jnp.int32),
            pltpu.VMEM((2 * WCH, CHUNK), jnp.int32),
            [pltpu.VMEM((CHUNK, W), _f32) for _ in range(G)],
            pltpu.VMEM_SHARED((N_PAD, W), _f32),
            pltpu.SemaphoreType.DMA,
            pltpu.SemaphoreType.DMA,
            pltpu.SemaphoreType.DMA,
        ),
    )
    def seg(tab, idx3, out, ibufa, ibufb, bufs, acc, gsem, ssem, isem):
        cid = lax.axis_index("c")
        sid = lax.axis_index("s")
        wid = sid * NC + cid

        def _zrow(i, carry):
            for c in range(W // 16):
                bufs[0][i, pl.ds(c * 16, 16)] = jnp.zeros((16,), _f32)
            return carry
        lax.fori_loop(0, CHUNK, _zrow, 0)
        for r in range(RPS // CHUNK):
            off = sid * RPS + r * CHUNK
            pltpu.sync_copy(bufs[0], acc.at[pl.ds(off, CHUNK)])
        plsc.subcore_barrier()

        ibufs = (ibufa, ibufb)
        pltpu.sync_copy(idx3.at[wid, pl.ds(0, 2 * WCH)], ibufa)
        ihs = {}
        if NWIN > 1:
            ihs[1] = pltpu.async_copy(idx3.at[wid, pl.ds(2 * WCH, 2 * WCH)],
                                      ibufb, isem)
        idx_ready = {0: True}
        pref_done = {}

        def _srow(c, kind):
            v = c // WCH
            if kind == 0 and v not in idx_ready:
                ihs[v].wait()
                idx_ready[v] = True
            if kind == 1 and v not in pref_done:
                pref_done[v] = True
                nv = v + 1
                if nv < NWIN and nv not in ihs:
                    ihs[nv] = pltpu.async_copy(
                        idx3.at[wid, pl.ds(nv * 2 * WCH, 2 * WCH)],
                        ibufs[nv % 2], isem)
            return ibufs[v % 2].at[2 * (c % WCH) + kind]

        ghs = {
            0: pltpu.async_copy(tab.at[_srow(0, 0)], bufs[0], gsem),
            1: pltpu.async_copy(tab.at[_srow(1, 0)], bufs[1], gsem),
        }
        shs = {}
        for c in range(NCH):
            ghs[c].wait()
            shs[c] = pltpu.async_copy(bufs[c % G], acc.at[_srow(c, 1)],
                                      ssem, add=True)
            if c + G < NCH:
                shs[c].wait()
                ghs[c + G] = pltpu.async_copy(tab.at[_srow(c + G, 0)],
                                              bufs[c % G], gsem)
        for c in range(NCH - G, NCH):
            shs[c].wait()
        plsc.subcore_barrier()

        for r in range(RPS // CHUNK):
            off = sid * RPS + r * CHUNK
            pltpu.sync_copy(acc.at[pl.ds(off, CHUNK)],
                            out.at[pl.ds(cid * N_PAD + off, CHUNK)])

    return seg


_seg_sum_cache = {}


def _seg_sum(tab, idx3):
    if "k" not in _seg_sum_cache:
        _seg_sum_cache["k"] = _make_seg_sum()
    return _seg_sum_cache["k"](tab, idx3).reshape(NC, N_PAD, W)


def _prep_edges(ei):
    src = ei[0].astype(jnp.int32)
    dst = ei[1].astype(jnp.int32)
    npad = E_PAD - E
    padv = N + (jnp.arange(npad, dtype=jnp.int32) % (N_PAD - N))
    src_p = jnp.concatenate([src, padv]).reshape(NW, NCH, 1, CHUNK)
    dst_p = jnp.concatenate([dst, padv]).reshape(NW, NCH, 1, CHUNK)
    return jnp.concatenate([src_p, dst_p], 2).reshape(NW, NCH * 2, CHUNK)


def _pad_rows(x):
    return jnp.pad(x, ((0, N_PAD - N), (0, 0)))


@jax.jit
def kernel(x_user, x_item, edge_index_ui, edge_index_iu,
           W_l0_ui, b_l0_ui, W_r0_ui, W_l0_iu, b_l0_iu, W_r0_iu,
           W_l1_ui, b_l1_ui, W_r1_ui, W_l1_iu, b_l1_iu, W_r1_iu,
           W_lin_user, b_lin_user, W_lin_item, b_lin_item):
    idx_ui = _prep_edges(edge_index_ui)
    idx_iu = _prep_edges(edge_index_iu)
    xu = _pad_rows(x_user)
    xi = _pad_rows(x_item)
    b2 = lambda b: b.reshape(1, -1)

    yu, yi = pl.pallas_call(
        _tc1_body,
        grid=(GRID,),
        in_specs=[_row_spec(D_IN), _row_spec(D_IN),
                  _full_spec((H, D_IN)), _full_spec((H, D_IN))],
        out_specs=[_row_spec(W), _row_spec(W)],
        out_shape=[jax.ShapeDtypeStruct((N_PAD, W), _f32)] * 2,
    )(xu, xi, W_l0_ui, W_l0_iu)

    p_item = _seg_sum(yu, idx_ui)
    p_user = _seg_sum(yi, idx_iu)

    hi, hu, ti, tu = pl.pallas_call(
        _tc2_body,
        grid=(GRID,),
        in_specs=[_part_spec(W), _part_spec(W),
                  _row_spec(D_IN), _row_spec(D_IN),
                  _full_spec((H, D_IN)), _full_spec((1, H)),
                  _full_spec((H, D_IN)), _full_spec((1, H)),
                  _full_spec((D_EMB, H)), _full_spec((1, D_EMB)),
                  _full_spec((D_EMB, H)), _full_spec((1, D_EMB))],
        out_specs=[_row_spec(W), _row_spec(W),
                   _row_spec(D_EMB), _row_spec(D_EMB)],
        out_shape=[jax.ShapeDtypeStruct((N_PAD, W), _f32)] * 2 +
                  [jax.ShapeDtypeStruct((N_PAD, D_EMB), _f32)] * 2,
    )(p_item, p_user, xi, xu,
      W_r0_ui, b2(b_l0_ui), W_r0_iu, b2(b_l0_iu),
      W_r1_ui, b2(b_l1_ui), W_r1_iu, b2(b_l1_iu))

    p1_item = _seg_sum(hu, idx_ui)
    p1_user = _seg_sum(hi, idx_iu)

    oi, ou = pl.pallas_call(
        _tc3_body,
        grid=(GRID,),
        in_specs=[_part_spec(W), _part_spec(W),
                  _part_spec(W), _part_spec(W),
                  _row_spec(D_EMB), _row_spec(D_EMB),
                  _full_spec((D_EMB, H)), _full_spec((D_EMB, H)),
                  _full_spec((D_EMB, D_EMB)), _full_spec((1, D_EMB)),
                  _full_spec((D_EMB, D_EMB)), _full_spec((1, D_EMB))],
        out_specs=[_row_spec(D_EMB), _row_spec(D_EMB)],
        out_shape=[jax.ShapeDtypeStruct((N_PAD, D_EMB), _f32)] * 2,
    )(p1_item, p1_user, p_item, p_user, ti, tu,
      W_l1_ui, W_l1_iu, W_lin_item, b2(b_lin_item),
      W_lin_user, b2(b_lin_user))

    return (ou[:N], oi[:N])
